# speedup vs baseline: 2.2759x; 2.2759x over previous
import jax
import jax.numpy as jnp
from jax import lax
from jax.experimental import pallas as pl
from jax.experimental.pallas import tpu as pltpu
from jax.experimental.pallas import tpu_sc as plsc

_NW = 32
_CHUNK = 128
_CHUNKS_PER_W = 64
_D = 128


def _embed_body(table_hbm, idx_hbm, out_hbm, idx_v, rows_v, sem):
    c = lax.axis_index("c")
    s = lax.axis_index("s")
    wid = s * 2 + c
    pltpu.sync_copy(idx_hbm.at[wid], idx_v)

    def step(j, carry):
        pltpu.async_copy(table_hbm.at[idx_v.at[j]], rows_v, sem).wait()
        pltpu.sync_copy(rows_v, out_hbm.at[wid, j])
        return carry

    lax.fori_loop(0, _CHUNKS_PER_W, step, 0)


def kernel(species, W):
    idx = species.reshape(_NW, _CHUNKS_PER_W, _CHUNK)
    mesh = plsc.VectorSubcoreMesh(core_axis_name="c", subcore_axis_name="s")
    k = pl.kernel(
        _embed_body,
        out_type=jax.ShapeDtypeStruct((_NW, _CHUNKS_PER_W, _CHUNK, _D), jnp.float32),
        mesh=mesh,
        scratch_types=[
            pltpu.VMEM((_CHUNKS_PER_W, _CHUNK), jnp.int32),
            pltpu.VMEM((_CHUNK, _D), jnp.float32),
            pltpu.SemaphoreType.DMA,
        ],
    )
    out = k(W, idx)
    b, g = species.shape[0], species.shape[1]
    return out.reshape(b, g, g, g, _D)


# speedup vs baseline: 2.2982x; 1.0098x over previous
import jax
import jax.numpy as jnp
from jax import lax
from jax.experimental import pallas as pl
from jax.experimental.pallas import tpu as pltpu
from jax.experimental.pallas import tpu_sc as plsc

_NW = 32
_CHUNK = 128
_CHUNKS_PER_W = 64
_D = 128


def _embed_body(table_hbm, idx_hbm, out_hbm, idx_v, r0, r1, r2, r3, gsem, wsem):
    c = lax.axis_index("c")
    s = lax.axis_index("s")
    wid = s * 2 + c
    pltpu.sync_copy(idx_hbm.at[wid], idx_v)
    bufs = (r0, r1, r2, r3)

    pltpu.async_copy(table_hbm.at[idx_v.at[0]], r0, gsem.at[0])
    pltpu.async_copy(table_hbm.at[idx_v.at[1]], r1, gsem.at[1])

    def outer(i, carry):
        j0 = i * 4
        for b in range(4):
            j = j0 + b
            cur = bufs[b]
            pltpu.make_async_copy(table_hbm.at[idx_v.at[j]], cur, gsem.at[b]).wait()
            pltpu.async_copy(cur, out_hbm.at[wid, j], wsem.at[b])
            nb_slot = (b + 2) % 4
            nb = bufs[nb_slot]

            @pl.when(j + 2 < _CHUNKS_PER_W)
            def _():
                @pl.when(j >= 2)
                def _():
                    pltpu.make_async_copy(
                        nb, out_hbm.at[wid, j - 2], wsem.at[nb_slot]
                    ).wait()

                pltpu.async_copy(table_hbm.at[idx_v.at[j + 2]], nb, gsem.at[nb_slot])

        return carry

    lax.fori_loop(0, _CHUNKS_PER_W // 4, outer, 0)
    pltpu.make_async_copy(r2, out_hbm.at[wid, _CHUNKS_PER_W - 2], wsem.at[2]).wait()
    pltpu.make_async_copy(r3, out_hbm.at[wid, _CHUNKS_PER_W - 1], wsem.at[3]).wait()


def kernel(species, W):
    idx = species.reshape(_NW, _CHUNKS_PER_W, _CHUNK)
    mesh = plsc.VectorSubcoreMesh(core_axis_name="c", subcore_axis_name="s")
    k = pl.kernel(
        _embed_body,
        out_type=jax.ShapeDtypeStruct((_NW, _CHUNKS_PER_W, _CHUNK, _D), jnp.float32),
        mesh=mesh,
        scratch_types=[
            pltpu.VMEM((_CHUNKS_PER_W, _CHUNK), jnp.int32),
            pltpu.VMEM((_CHUNK, _D), jnp.float32),
            pltpu.VMEM((_CHUNK, _D), jnp.float32),
            pltpu.VMEM((_CHUNK, _D), jnp.float32),
            pltpu.VMEM((_CHUNK, _D), jnp.float32),
            pltpu.SemaphoreType.DMA((4,)),
            pltpu.SemaphoreType.DMA((4,)),
        ],
    )
    out = k(W, idx)
    b, g = species.shape[0], species.shape[1]
    return out.reshape(b, g, g, g, _D)


# speedup vs baseline: 12.5254x; 5.4500x over previous
import jax
import jax.numpy as jnp
from jax import lax
from jax.experimental import pallas as pl
from jax.experimental.pallas import tpu as pltpu
from jax.experimental.pallas import tpu_sc as plsc

_NW = 32
_CHUNK = 128
_CHUNKS_PER_W = 64
_D = 128


def _embed_body(table_hbm, idx_hbm, out_hbm, table_v, idx_v, r0, r1, r2, r3, gsem, wsem):
    c = lax.axis_index("c")
    s = lax.axis_index("s")
    wid = s * 2 + c
    pltpu.sync_copy(table_hbm, table_v)
    pltpu.sync_copy(idx_hbm.at[wid], idx_v)
    bufs = (r0, r1, r2, r3)

    pltpu.async_copy(table_v.at[idx_v.at[0]], r0, gsem.at[0])
    pltpu.async_copy(table_v.at[idx_v.at[1]], r1, gsem.at[1])

    def outer(i, carry):
        j0 = i * 4
        for b in range(4):
            j = j0 + b
            cur = bufs[b]
            pltpu.make_async_copy(table_v.at[idx_v.at[j]], cur, gsem.at[b]).wait()
            pltpu.async_copy(cur, out_hbm.at[wid, j], wsem.at[b])
            nb_slot = (b + 2) % 4
            nb = bufs[nb_slot]

            @pl.when(j + 2 < _CHUNKS_PER_W)
            def _():
                @pl.when(j >= 2)
                def _():
                    pltpu.make_async_copy(
                        nb, out_hbm.at[wid, j - 2], wsem.at[nb_slot]
                    ).wait()

                pltpu.async_copy(table_v.at[idx_v.at[j + 2]], nb, gsem.at[nb_slot])

        return carry

    lax.fori_loop(0, _CHUNKS_PER_W // 4, outer, 0)
    pltpu.make_async_copy(r2, out_hbm.at[wid, _CHUNKS_PER_W - 2], wsem.at[2]).wait()
    pltpu.make_async_copy(r3, out_hbm.at[wid, _CHUNKS_PER_W - 1], wsem.at[3]).wait()


def kernel(species, W):
    idx = species.reshape(_NW, _CHUNKS_PER_W, _CHUNK)
    mesh = plsc.VectorSubcoreMesh(core_axis_name="c", subcore_axis_name="s")
    k = pl.kernel(
        _embed_body,
        out_type=jax.ShapeDtypeStruct((_NW, _CHUNKS_PER_W, _CHUNK, _D), jnp.float32),
        mesh=mesh,
        scratch_types=[
            pltpu.VMEM_SHARED((92, _D), jnp.float32),
            pltpu.VMEM((_CHUNKS_PER_W, _CHUNK), jnp.int32),
            pltpu.VMEM((_CHUNK, _D), jnp.float32),
            pltpu.VMEM((_CHUNK, _D), jnp.float32),
            pltpu.VMEM((_CHUNK, _D), jnp.float32),
            pltpu.VMEM((_CHUNK, _D), jnp.float32),
            pltpu.SemaphoreType.DMA((4,)),
            pltpu.SemaphoreType.DMA((4,)),
        ],
    )
    out = k(W, idx)
    b, g = species.shape[0], species.shape[1]
    return out.reshape(b, g, g, g, _D)


# speedup vs baseline: 12.5335x; 1.0006x over previous
import jax
import jax.numpy as jnp
from jax import lax
from jax.experimental import pallas as pl
from jax.experimental.pallas import tpu as pltpu
from jax.experimental.pallas import tpu_sc as plsc

_NW = 32
_CHUNK = 128
_CHUNKS_PER_W = 64
_D = 128


def _embed_body(table_hbm, idx_hbm, out_hbm, table_v, idx_v, r0, r1, r2, r3, gsem, wsem):
    c = lax.axis_index("c")
    s = lax.axis_index("s")
    wid = s * 2 + c
    pltpu.sync_copy(table_hbm, table_v)
    pltpu.sync_copy(idx_hbm.at[wid], idx_v)
    bufs = (r0, r1, r2, r3)

    pltpu.async_copy(table_v.at[idx_v.at[0]], r0, gsem.at[0])
    pltpu.async_copy(table_v.at[idx_v.at[1]], r1, gsem.at[1])
    pltpu.async_copy(table_v.at[idx_v.at[2]], r2, gsem.at[2])

    def outer(i, carry):
        j0 = i * 4
        for b in range(4):
            j = j0 + b
            cur = bufs[b]
            pltpu.make_async_copy(table_v.at[idx_v.at[j]], cur, gsem.at[b]).wait()
            pltpu.async_copy(cur, out_hbm.at[wid, j], wsem.at[b])
            nb_slot = (b + 3) % 4
            nb = bufs[nb_slot]

            @pl.when(j + 3 < _CHUNKS_PER_W)
            def _():
                @pl.when(j >= 1)
                def _():
                    pltpu.make_async_copy(
                        nb, out_hbm.at[wid, j - 1], wsem.at[nb_slot]
                    ).wait()

                pltpu.async_copy(table_v.at[idx_v.at[j + 3]], nb, gsem.at[nb_slot])

        return carry

    lax.fori_loop(0, _CHUNKS_PER_W // 4, outer, 0)
    pltpu.make_async_copy(r0, out_hbm.at[wid, _CHUNKS_PER_W - 4], wsem.at[0]).wait()
    pltpu.make_async_copy(r1, out_hbm.at[wid, _CHUNKS_PER_W - 3], wsem.at[1]).wait()
    pltpu.make_async_copy(r2, out_hbm.at[wid, _CHUNKS_PER_W - 2], wsem.at[2]).wait()
    pltpu.make_async_copy(r3, out_hbm.at[wid, _CHUNKS_PER_W - 1], wsem.at[3]).wait()


def kernel(species, W):
    idx = species.reshape(_NW, _CHUNKS_PER_W, _CHUNK)
    mesh = plsc.VectorSubcoreMesh(core_axis_name="c", subcore_axis_name="s")
    k = pl.kernel(
        _embed_body,
        out_type=jax.ShapeDtypeStruct((_NW, _CHUNKS_PER_W, _CHUNK, _D), jnp.float32),
        mesh=mesh,
        scratch_types=[
            pltpu.VMEM_SHARED((92, _D), jnp.float32),
            pltpu.VMEM((_CHUNKS_PER_W, _CHUNK), jnp.int32),
            pltpu.VMEM((_CHUNK, _D), jnp.float32),
            pltpu.VMEM((_CHUNK, _D), jnp.float32),
            pltpu.VMEM((_CHUNK, _D), jnp.float32),
            pltpu.VMEM((_CHUNK, _D), jnp.float32),
            pltpu.SemaphoreType.DMA((4,)),
            pltpu.SemaphoreType.DMA((4,)),
        ],
    )
    out = k(W, idx)
    b, g = species.shape[0], species.shape[1]
    return out.reshape(b, g, g, g, _D)
